# baseline (speedup 1.0000x reference)
import functools
import math

import jax
import jax.numpy as jnp
from jax import lax
from jax.experimental import pallas as pl
from jax.experimental.pallas import tpu as pltpu
from jax.experimental.pallas import tpu_sc as plsc

D_MODEL = 64
SCALE = math.sqrt(D_MODEL)

NC = 2
NS = 16
NW = NC * NS
LANES = 16

SUB = 128
NSUB = 8
CH = SUB * NSUB


@functools.lru_cache(maxsize=None)
def _make_kernel(B: int, V: int):
    rows_per_w = B // NW
    n_chunks = rows_per_w // CH
    mesh = plsc.VectorSubcoreMesh(core_axis_name="c", subcore_axis_name="s")

    @functools.partial(
        pl.kernel,
        mesh=mesh,
        out_type=jax.ShapeDtypeStruct((B, D_MODEL), jnp.float32),
        scratch_types=[
            pltpu.VMEM((NSUB, SUB), jnp.int32),
            pltpu.VMEM((CH, D_MODEL), jnp.float32),
            pltpu.SemaphoreType.DMA,
        ],
        compiler_params=pltpu.CompilerParams(use_tc_tiling_on_sc=False),
    )
    def emb_kernel(x_hbm, table_hbm, out_hbm, idx_v, rows_v, sem):
        wid = lax.axis_index("s") * NC + lax.axis_index("c")
        base = wid * rows_per_w

        def chunk_body(g, carry):
            row0 = pl.multiple_of(base + g * CH, CH)
            pltpu.sync_copy(
                x_hbm.at[pl.ds(pl.multiple_of(row0 // SUB, NSUB), NSUB)], idx_v
            )
            copies = []
            for j in range(NSUB):
                copies.append(
                    pltpu.async_copy(
                        table_hbm.at[idx_v.at[j]],
                        rows_v.at[pl.ds(j * SUB, SUB)],
                        sem,
                    )
                )
            for c in copies:
                c.wait()

            def scale_row(i, c2):
                for j in range(D_MODEL // LANES):
                    sl = (i, pl.ds(j * LANES, LANES))
                    rows_v[sl] = rows_v[sl] * SCALE
                return c2

            lax.fori_loop(0, CH, scale_row, 0)

            pltpu.sync_copy(rows_v, out_hbm.at[pl.ds(row0, CH)])
            return carry

        lax.fori_loop(0, n_chunks, chunk_body, 0)

    return emb_kernel


def kernel(x, emb_weight):
    B = x.shape[0] * x.shape[1]
    V = emb_weight.shape[0]
    xf = x.reshape(B // SUB, SUB).astype(jnp.int32)
    out = _make_kernel(B, V)(xf, emb_weight)
    return out.reshape(x.shape[0], x.shape[1], D_MODEL)


# speedup vs baseline: 1.1099x; 1.1099x over previous
import functools
import math

import jax
import jax.numpy as jnp
from jax import lax
from jax.experimental import pallas as pl
from jax.experimental.pallas import tpu as pltpu
from jax.experimental.pallas import tpu_sc as plsc

D_MODEL = 64
SCALE = math.sqrt(D_MODEL)

NC = 2
NS = 16
NW = NC * NS
LANES = 16

SUB = 128
H = 256
GPH = H // SUB
NB = 4


@functools.lru_cache(maxsize=None)
def _make_kernel(B: int):
    rows_per_w = B // NW
    idx_rows = rows_per_w // SUB
    nh = rows_per_w // H
    assert nh % NB == 0 and idx_rows % 8 == 0
    mesh = plsc.VectorSubcoreMesh(core_axis_name="c", subcore_axis_name="s")

    @functools.partial(
        pl.kernel,
        mesh=mesh,
        out_type=jax.ShapeDtypeStruct((B, D_MODEL), jnp.float32),
        scratch_types=[
            pltpu.VMEM((idx_rows, SUB), jnp.int32),
            pltpu.VMEM((NB, H, D_MODEL), jnp.float32),
            [pltpu.SemaphoreType.DMA] * NB,
            [pltpu.SemaphoreType.DMA] * NB,
        ],
        compiler_params=pltpu.CompilerParams(use_tc_tiling_on_sc=False),
    )
    def emb_kernel(x_hbm, table_hbm, out_hbm, idx_v, rows_v, gsems, osems):
        wid = lax.axis_index("s") * NC + lax.axis_index("c")
        base = wid * rows_per_w

        pltpu.sync_copy(
            x_hbm.at[pl.ds(pl.multiple_of(wid * idx_rows, 8), idx_rows)], idx_v
        )

        def gather_fire(h, b):
            for k in range(GPH):
                pltpu.async_copy(
                    table_hbm.at[idx_v.at[h * GPH + k]],
                    rows_v.at[b, pl.ds(k * SUB, SUB)],
                    gsems[b],
                )

        def gather_wait(b):
            for k in range(GPH):
                pltpu.make_async_copy(
                    table_hbm.at[idx_v.at[k]],
                    rows_v.at[b, pl.ds(k * SUB, SUB)],
                    gsems[b],
                ).wait()

        def out_row0(h):
            return pl.multiple_of(base + h * H, H)

        def out_fire(h, b):
            pltpu.async_copy(
                rows_v.at[b], out_hbm.at[pl.ds(out_row0(h), H)], osems[b]
            )

        def out_wait(h, b):
            pltpu.make_async_copy(
                rows_v.at[b], out_hbm.at[pl.ds(out_row0(h), H)], osems[b]
            ).wait()

        def scale(b):
            buf = rows_v.at[b]

            @plsc.parallel_loop(0, H, unroll=8)
            def _(i):
                for j in range(D_MODEL // LANES):
                    sl = (i, pl.ds(j * LANES, LANES))
                    buf[sl] = buf[sl] * SCALE

        gather_fire(0, 0)

        @pl.loop(0, nh, step=NB)
        def steps(h0):
            for b in range(NB):
                h = h0 + b
                nxt = (b + 1) % NB
                if b == NB - 1:
                    out_wait(h - (NB - 1), nxt)
                else:
                    @pl.when(h0 > 0)
                    def _():
                        out_wait(h - (NB - 1), nxt)
                if b == NB - 1:
                    @pl.when(h0 < nh - NB)
                    def _():
                        gather_fire(h + 1, nxt)
                else:
                    gather_fire(h + 1, nxt)
                gather_wait(b)
                scale(b)
                out_fire(h, b)

        for b in range(1, NB):
            out_wait(nh - NB + b, b)

    return emb_kernel


def kernel(x, emb_weight):
    B = x.shape[0] * x.shape[1]
    xf = x.reshape(B // SUB, SUB).astype(jnp.int32)
    out = _make_kernel(B)(xf, emb_weight)
    return out.reshape(x.shape[0], x.shape[1], D_MODEL)
